# baseline (speedup 1.0000x reference)
import functools

import jax
import jax.numpy as jnp
from jax import lax
from jax.experimental import pallas as pl
from jax.experimental.pallas import tpu as pltpu
from jax.experimental.pallas import tpu_sc as plsc

_LANES = 16
_NC = 2
_NS = 16
_NW = _NC * _NS
_CH = 4096
_G1 = 16
_SCH = 512
_NSB = 5
_G2 = 8


def _copy_body(a_ref, o_ref):
    o_ref[...] = a_ref[...]


def _xpose_body(idx_ref, src_ref, idxt_ref, srct_ref):
    idxt_ref[...] = idx_ref[...].T
    srct_ref[...] = src_ref[...].T


@functools.cache
def _make_sc_scatter(m, b, d):
    cpw = d // _NW
    n_chunks = b // _CH
    g1_iters = _CH // (_LANES * _G1)
    n_sub = _CH // _SCH
    g2_iters = _SCH // (_LANES * _G2)
    mesh = plsc.VectorSubcoreMesh(core_axis_name="c", subcore_axis_name="s")

    @functools.partial(
        pl.kernel,
        mesh=mesh,
        compiler_params=pltpu.CompilerParams(needs_layout_passes=False),
        scratch_types=[
            pltpu.VMEM((m,), jnp.int32),
            pltpu.VMEM((b,), jnp.float32),
            pltpu.VMEM((_CH,), jnp.int32),
            pltpu.VMEM((_CH,), jnp.int32),
        ] + [pltpu.VMEM((_SCH,), jnp.int32) for _ in range(_NSB)]
          + [pltpu.VMEM((_SCH,), jnp.float32) for _ in range(_NSB)]
          + [pltpu.SemaphoreType.DMA for _ in range(2 + _NSB)],
    )
    def sc_scatter(out_ref, idxt_hbm, srct_hbm,
                   win_ref, srcc_ref, idx_a, idx_b, *rest):
        off_bufs = rest[:_NSB]
        val_bufs = rest[_NSB:2 * _NSB]
        sem_i = rest[2 * _NSB:2 * _NSB + 2]
        sem_s = rest[2 * _NSB + 2:]
        idx_bufs = (idx_a, idx_b)
        cc = lax.axis_index("c")
        ss = lax.axis_index("s")
        wid = ss * _NC + cc
        lanes = lax.iota(jnp.int32, _LANES)

        def column(col, _):
            j = wid * cpw + col
            with jax.named_scope("src_col_load"):
                pltpu.sync_copy(srct_hbm.at[j], srcc_ref)

            def load_idx(c):
                return pltpu.async_copy(
                    idxt_hbm.at[j, pl.ds(c * _CH, _CH)],
                    idx_bufs[c % 2], sem_i[c % 2])

            ns1 = jax.named_scope("pass1")
            ns1.__enter__()
            idesc = load_idx(0)
            for c in range(n_chunks):
                nxt = load_idx(c + 1) if c + 1 < n_chunks else None
                idesc.wait()
                cb = idx_bufs[c % 2]

                def p1_group(gi, _, c=c, cb=cb):
                    base = c * _CH + gi * (_LANES * _G1)
                    ts = []
                    ivs = []
                    for v in range(_G1):
                        t = cb[pl.ds(gi * (_LANES * _G1) + v * _LANES, _LANES)]
                        iv = base + v * _LANES + lanes
                        plsc.store_scatter(win_ref, [t], iv)
                        ts.append(t)
                        ivs.append(iv)
                    bad = None
                    for v in range(_G1):
                        w = plsc.load_gather(win_ref, [ts[v]])
                        mv = w < ivs[v]
                        bad = mv if bad is None else (bad | mv)

                    def fix_cond(nbad):
                        return nbad > 0

                    def fix_body(nbad):
                        accv = jnp.zeros((_LANES,), jnp.int32)
                        for v in range(_G1):
                            w = plsc.load_gather(win_ref, [ts[v]])
                            mv = w < ivs[v]
                            plsc.store_scatter(win_ref, [ts[v]], ivs[v], mask=mv)
                        for v in range(_G1):
                            w = plsc.load_gather(win_ref, [ts[v]])
                            accv = accv + (w < ivs[v]).astype(jnp.int32)
                        return jnp.sum(accv)

                    lax.while_loop(fix_cond, fix_body,
                                   jnp.any(bad).astype(jnp.int32))
                    return 0

                lax.fori_loop(0, g1_iters, p1_group, 0)
                idesc = nxt

            ns1.__exit__(None, None, None)
            ns2 = jax.named_scope("pass2")
            ns2.__enter__()
            sdescs = [None] * _NSB
            seq = 0
            idesc = load_idx(0)
            for c in range(n_chunks):
                nxt = load_idx(c + 1) if c + 1 < n_chunks else None
                idesc.wait()
                cb = idx_bufs[c % 2]
                for sub in range(n_sub):
                    sb = seq % _NSB
                    seq += 1
                    if sdescs[sb] is not None:
                        sdescs[sb].wait()
                    ob, vb = off_bufs[sb], val_bufs[sb]

                    def p2_group(gi, _, cb=cb, sub=sub, ob=ob, vb=vb):
                        for v in range(_G2):
                            sl_in = pl.ds(sub * _SCH
                                          + gi * (_LANES * _G2) + v * _LANES,
                                          _LANES)
                            sl_out = pl.ds(gi * (_LANES * _G2) + v * _LANES,
                                           _LANES)
                            t = cb[sl_in]
                            w = plsc.load_gather(win_ref, [t])
                            vals = plsc.load_gather(srcc_ref, [w])
                            ob[sl_out] = t * d + j
                            vb[sl_out] = vals
                        return 0

                    lax.fori_loop(0, g2_iters, p2_group, 0)
                    sdescs[sb] = pltpu.async_copy(
                        vb, out_ref.at[ob], sem_s[sb])
                idesc = nxt
            for sd in sdescs:
                if sd is not None:
                    sd.wait()
            ns2.__exit__(None, None, None)
            return 0

        lax.fori_loop(0, cpw, column, 0)

    return sc_scatter


def kernel(self_tensor, index, src):
    m, d = self_tensor.shape
    b = index.shape[0]
    n = m * d
    copy_grid = 25
    xpose_grid = 8

    a_flat = self_tensor.reshape(n)
    out0 = pl.pallas_call(
        _copy_body,
        grid=(copy_grid,),
        in_specs=[pl.BlockSpec((n // copy_grid,), lambda g: (g,))],
        out_specs=pl.BlockSpec((n // copy_grid,), lambda g: (g,)),
        out_shape=jax.ShapeDtypeStruct((n,), jnp.float32),
    )(a_flat)

    idxt, srct = pl.pallas_call(
        _xpose_body,
        grid=(xpose_grid,),
        in_specs=[
            pl.BlockSpec((b // xpose_grid, d), lambda g: (g, 0)),
            pl.BlockSpec((b // xpose_grid, d), lambda g: (g, 0)),
        ],
        out_specs=[
            pl.BlockSpec((d, b // xpose_grid), lambda g: (0, g)),
            pl.BlockSpec((d, b // xpose_grid), lambda g: (0, g)),
        ],
        out_shape=[
            jax.ShapeDtypeStruct((d, b), jnp.int32),
            jax.ShapeDtypeStruct((d, b), jnp.float32),
        ],
    )(index, src)

    out_ref = jax.new_ref(out0)
    _make_sc_scatter(m, b, d)(out_ref, idxt, srct)
    return jax.freeze(out_ref).reshape(m, d)


# speedup vs baseline: 7.0658x; 7.0658x over previous
import functools

import jax
import jax.numpy as jnp
from jax import lax
from jax.experimental import pallas as pl
from jax.experimental.pallas import tpu as pltpu
from jax.experimental.pallas import tpu_sc as plsc

_LANES = 16
_NC = 2
_NS = 16
_NW = _NC * _NS
_CH = 4096
_G1 = 16
_G2 = 8


def _xpose_body(idx_ref, src_ref, idxt_ref, srct_ref):
    idxt_ref[...] = idx_ref[...].T
    srct_ref[...] = src_ref[...].T


def _merge_body(win2_ref, self_ref, out_ref, scr_ref):
    d, mb = self_ref.shape[1], self_ref.shape[0]
    for j in range(d):
        scr_ref[j, :] = win2_ref[pl.ds(j * mb, mb)]
    wt = scr_ref[...].T
    out_ref[...] = jnp.where(wt == -1, self_ref[...],
                             jax.lax.bitcast_convert_type(wt, jnp.float32))


_NP = 10


@functools.cache
def _make_sc_winner(m, b, d):
    cpw = d // _NW
    n_chunks = b // _CH
    g1_iters = _CH // (_LANES * _G1)
    g2_iters = _CH // (_LANES * _G2)
    m_vregs = m // _LANES
    pm = m // _NP
    mesh = plsc.VectorSubcoreMesh(core_axis_name="c", subcore_axis_name="s")

    @functools.partial(
        pl.kernel,
        out_type=jax.ShapeDtypeStruct((_NP * d * pm,), jnp.int32),
        mesh=mesh,
        compiler_params=pltpu.CompilerParams(needs_layout_passes=False),
        scratch_types=[
            pltpu.VMEM((m,), jnp.int32),
            pltpu.VMEM((_CH,), jnp.int32),
            pltpu.VMEM((_CH,), jnp.int32),
            pltpu.VMEM((_CH,), jnp.float32),
            pltpu.VMEM((_CH,), jnp.float32),
            pltpu.SemaphoreType.DMA,
            pltpu.SemaphoreType.DMA,
            pltpu.SemaphoreType.DMA,
            pltpu.SemaphoreType.DMA,
            pltpu.SemaphoreType.DMA,
        ],
    )
    def sc_winner(idxt_hbm, srct_hbm, win2_hbm,
                  win_ref, idx_a, idx_b, src_a, src_b,
                  sem_i0, sem_i1, sem_v0, sem_v1, sem_w):
        idx_bufs = (idx_a, idx_b)
        src_bufs = (src_a, src_b)
        sem_i = (sem_i0, sem_i1)
        sem_v = (sem_v0, sem_v1)
        cc = lax.axis_index("c")
        ss = lax.axis_index("s")
        wid = ss * _NC + cc
        lanes = lax.iota(jnp.int32, _LANES)
        neg1 = jnp.full((_LANES,), -1, jnp.int32)

        def column(col, _):
            j = wid * cpw + col

            def load_idx(c):
                return pltpu.async_copy(
                    idxt_hbm.at[j, pl.ds(c * _CH, _CH)],
                    idx_bufs[c % 2], sem_i[c % 2])

            def load_src(c):
                return pltpu.async_copy(
                    srct_hbm.at[j, pl.ds(c * _CH, _CH)],
                    src_bufs[c % 2], sem_v[c % 2])

            with jax.named_scope("meminit"):
                idesc = load_idx(0)

                def init_vreg(v, _):
                    win_ref[pl.ds(v * _LANES, _LANES)] = neg1
                    return 0

                lax.fori_loop(0, m_vregs, init_vreg, 0)

            ns1 = jax.named_scope("pass1")
            ns1.__enter__()
            for c in range(n_chunks):
                nxt = load_idx(c + 1) if c + 1 < n_chunks else None
                idesc.wait()
                cb = idx_bufs[c % 2]

                def p1_group(gi, _, c=c, cb=cb):
                    base = c * _CH + gi * (_LANES * _G1)
                    ts = []
                    ivs = []
                    for v in range(_G1):
                        t = cb[pl.ds(gi * (_LANES * _G1) + v * _LANES,
                                     _LANES)]
                        iv = base + v * _LANES + lanes
                        plsc.store_scatter(win_ref, [t], iv)
                        ts.append(t)
                        ivs.append(iv)
                    bad = None
                    for v in range(_G1):
                        w = plsc.load_gather(win_ref, [ts[v]])
                        mv = w < ivs[v]
                        bad = mv if bad is None else (bad | mv)

                    def fix_cond(nbad):
                        return nbad > 0

                    def fix_body(nbad):
                        accv = jnp.zeros((_LANES,), jnp.int32)
                        for v in range(_G1):
                            w = plsc.load_gather(win_ref, [ts[v]])
                            plsc.store_scatter(win_ref, [ts[v]], ivs[v],
                                               mask=w < ivs[v])
                        for v in range(_G1):
                            w = plsc.load_gather(win_ref, [ts[v]])
                            accv = accv + (w < ivs[v]).astype(jnp.int32)
                        return jnp.sum(accv)

                    lax.while_loop(fix_cond, fix_body,
                                   jnp.any(bad).astype(jnp.int32))
                    return 0

                lax.fori_loop(0, g1_iters, p1_group, 0)
                idesc = nxt
            ns1.__exit__(None, None, None)

            ns2 = jax.named_scope("pass2")
            ns2.__enter__()
            idesc = load_idx(0)
            vdesc = load_src(0)
            for c in range(n_chunks):
                nxti = load_idx(c + 1) if c + 1 < n_chunks else None
                nxtv = load_src(c + 1) if c + 1 < n_chunks else None
                idesc.wait()
                vdesc.wait()
                cb = idx_bufs[c % 2]
                vb = src_bufs[c % 2]

                def p2_group(gi, _, c=c, cb=cb, vb=vb):
                    base = c * _CH + gi * (_LANES * _G2)
                    for v in range(_G2):
                        sl = pl.ds(gi * (_LANES * _G2) + v * _LANES, _LANES)
                        t = cb[sl]
                        iv = base + v * _LANES + lanes
                        w = plsc.load_gather(win_ref, [t])
                        vals = plsc.bitcast(vb[sl], jnp.int32)
                        plsc.store_scatter(win_ref, [t], vals, mask=w == iv)
                    return 0

                lax.fori_loop(0, g2_iters, p2_group, 0)
                idesc = nxti
                vdesc = nxtv
            ns2.__exit__(None, None, None)

            with jax.named_scope("win_out"):
                wdescs = [
                    pltpu.async_copy(win_ref.at[pl.ds(p * pm, pm)],
                                     win2_hbm.at[pl.ds((p * d + j) * pm, pm)],
                                     sem_w)
                    for p in range(_NP)
                ]
                for wd in wdescs:
                    wd.wait()
            return 0

        lax.fori_loop(0, cpw, column, 0)

    return sc_winner


def kernel(self_tensor, index, src):
    m, d = self_tensor.shape
    b = index.shape[0]
    xpose_grid = 8
    merge_grid = _NP
    mb = m // merge_grid

    idxt, srct = pl.pallas_call(
        _xpose_body,
        grid=(xpose_grid,),
        in_specs=[
            pl.BlockSpec((b // xpose_grid, d), lambda g: (g, 0)),
            pl.BlockSpec((b // xpose_grid, d), lambda g: (g, 0)),
        ],
        out_specs=[
            pl.BlockSpec((d, b // xpose_grid), lambda g: (0, g)),
            pl.BlockSpec((d, b // xpose_grid), lambda g: (0, g)),
        ],
        out_shape=[
            jax.ShapeDtypeStruct((d, b), jnp.int32),
            jax.ShapeDtypeStruct((d, b), jnp.float32),
        ],
    )(index, src)

    win2 = _make_sc_winner(m, b, d)(idxt, srct)

    out = pl.pallas_call(
        _merge_body,
        grid=(merge_grid,),
        in_specs=[
            pl.BlockSpec((d * mb,), lambda g: (g,)),
            pl.BlockSpec((mb, d), lambda g: (g, 0)),
        ],
        out_specs=pl.BlockSpec((mb, d), lambda g: (g, 0)),
        out_shape=jax.ShapeDtypeStruct((m, d), jnp.float32),
        scratch_shapes=[pltpu.VMEM((d, mb), jnp.int32)],
    )(win2, self_tensor)
    return out


# speedup vs baseline: 14.5220x; 2.0552x over previous
import functools

import jax
import jax.numpy as jnp
from jax import lax
from jax.experimental import pallas as pl
from jax.experimental.pallas import tpu as pltpu
from jax.experimental.pallas import tpu_sc as plsc

_LANES = 16
_NC = 2
_NS = 16
_NW = _NC * _NS
_CH = 4096
_G1 = 16
_G2 = 16


def _xpose_body(idx_ref, src_ref, idxt_ref, srct_ref):
    idxt_ref[...] = idx_ref[...].T
    srct_ref[...] = src_ref[...].T


def _merge_body(win2_ref, self_ref, out_ref, scr_ref):
    d, mb = self_ref.shape[1], self_ref.shape[0]
    for j in range(d):
        scr_ref[j, :] = win2_ref[pl.ds(j * mb, mb)]
    wt = scr_ref[...].T
    out_ref[...] = jnp.where(wt == -1, self_ref[...],
                             jax.lax.bitcast_convert_type(wt, jnp.float32))


_NP = 10


@functools.cache
def _make_sc_winner(m, b, d):
    cpw = d // _NW
    n_chunks = b // _CH
    g1_iters = _CH // (_LANES * _G1)
    g2_iters = _CH // (_LANES * _G2)
    m_vregs = m // _LANES
    pm = m // _NP
    mesh = plsc.VectorSubcoreMesh(core_axis_name="c", subcore_axis_name="s")

    @functools.partial(
        pl.kernel,
        out_type=jax.ShapeDtypeStruct((_NP * d * pm,), jnp.int32),
        mesh=mesh,
        compiler_params=pltpu.CompilerParams(needs_layout_passes=False),
        scratch_types=[
            pltpu.VMEM((m,), jnp.int32),
            pltpu.VMEM((_CH,), jnp.int32),
            pltpu.VMEM((_CH,), jnp.int32),
            pltpu.VMEM((_CH,), jnp.float32),
            pltpu.VMEM((_CH,), jnp.float32),
            pltpu.SemaphoreType.DMA,
            pltpu.SemaphoreType.DMA,
            pltpu.SemaphoreType.DMA,
            pltpu.SemaphoreType.DMA,
            pltpu.SemaphoreType.DMA,
        ],
    )
    def sc_winner(idxt_hbm, srct_hbm, win2_hbm,
                  win_ref, idx_a, idx_b, src_a, src_b,
                  sem_i0, sem_i1, sem_v0, sem_v1, sem_w):
        idx_bufs = (idx_a, idx_b)
        src_bufs = (src_a, src_b)
        sem_i = (sem_i0, sem_i1)
        sem_v = (sem_v0, sem_v1)
        cc = lax.axis_index("c")
        ss = lax.axis_index("s")
        wid = ss * _NC + cc
        lanes = lax.iota(jnp.int32, _LANES)
        neg1 = jnp.full((_LANES,), -1, jnp.int32)

        def column(col, _):
            j = wid * cpw + col

            def load_idx(c):
                return pltpu.async_copy(
                    idxt_hbm.at[j, pl.ds(c * _CH, _CH)],
                    idx_bufs[c % 2], sem_i[c % 2])

            def load_src(c):
                return pltpu.async_copy(
                    srct_hbm.at[j, pl.ds(c * _CH, _CH)],
                    src_bufs[c % 2], sem_v[c % 2])

            with jax.named_scope("meminit"):
                idesc = load_idx(0)
                unroll = 25

                def init_group(v, _):
                    for u in range(unroll):
                        win_ref[pl.ds((v * unroll + u) * _LANES, _LANES)] = \
                            neg1
                    return 0

                lax.fori_loop(0, m_vregs // unroll, init_group, 0)

            ns1 = jax.named_scope("pass1")
            ns1.__enter__()
            for c in range(n_chunks):
                nxt = load_idx(c + 1) if c + 1 < n_chunks else None
                idesc.wait()
                cb = idx_bufs[c % 2]

                def p1_group(gi, _, c=c, cb=cb):
                    base = c * _CH + gi * (_LANES * _G1)
                    ts = []
                    ivs = []
                    for v in range(_G1):
                        ts.append(cb[pl.ds(gi * (_LANES * _G1) + v * _LANES,
                                           _LANES)])
                        ivs.append(base + v * _LANES + lanes)
                    for v in range(_G1):
                        plsc.store_scatter(win_ref, [ts[v]], ivs[v])
                    bad = None
                    for v in range(_G1):
                        w = plsc.load_gather(win_ref, [ts[v]])
                        mv = w < ivs[v]
                        bad = mv if bad is None else (bad | mv)

                    def fix_cond(nbad):
                        return nbad > 0

                    def fix_body(nbad):
                        accv = jnp.zeros((_LANES,), jnp.int32)
                        for v in range(_G1):
                            w = plsc.load_gather(win_ref, [ts[v]])
                            plsc.store_scatter(win_ref, [ts[v]], ivs[v],
                                               mask=w < ivs[v])
                        for v in range(_G1):
                            w = plsc.load_gather(win_ref, [ts[v]])
                            accv = accv + (w < ivs[v]).astype(jnp.int32)
                        return jnp.sum(accv)

                    lax.while_loop(fix_cond, fix_body,
                                   jnp.any(bad).astype(jnp.int32))
                    return 0

                lax.fori_loop(0, g1_iters, p1_group, 0)
                idesc = nxt
            ns1.__exit__(None, None, None)

            ns2 = jax.named_scope("pass2")
            ns2.__enter__()
            idesc = load_idx(0)
            vdesc = load_src(0)
            for c in range(n_chunks):
                nxti = load_idx(c + 1) if c + 1 < n_chunks else None
                nxtv = load_src(c + 1) if c + 1 < n_chunks else None
                idesc.wait()
                vdesc.wait()
                cb = idx_bufs[c % 2]
                vb = src_bufs[c % 2]

                def p2_group(gi, _, c=c, cb=cb, vb=vb):
                    base = c * _CH + gi * (_LANES * _G2)
                    ts, masks, vals = [], [], []
                    for v in range(_G2):
                        sl = pl.ds(gi * (_LANES * _G2) + v * _LANES, _LANES)
                        t = cb[sl]
                        iv = base + v * _LANES + lanes
                        w = plsc.load_gather(win_ref, [t])
                        ts.append(t)
                        masks.append(w == iv)
                        vals.append(plsc.bitcast(vb[sl], jnp.int32))
                    for v in range(_G2):
                        plsc.store_scatter(win_ref, [ts[v]], vals[v],
                                           mask=masks[v])
                    return 0

                lax.fori_loop(0, g2_iters, p2_group, 0)
                idesc = nxti
                vdesc = nxtv
            ns2.__exit__(None, None, None)

            with jax.named_scope("win_out"):
                wdescs = [
                    pltpu.async_copy(win_ref.at[pl.ds(p * pm, pm)],
                                     win2_hbm.at[pl.ds((p * d + j) * pm, pm)],
                                     sem_w)
                    for p in range(_NP)
                ]
                for wd in wdescs:
                    wd.wait()
            return 0

        lax.fori_loop(0, cpw, column, 0)

    return sc_winner


def kernel(self_tensor, index, src):
    m, d = self_tensor.shape
    b = index.shape[0]
    xpose_grid = 8
    merge_grid = _NP
    mb = m // merge_grid

    idxt, srct = pl.pallas_call(
        _xpose_body,
        grid=(xpose_grid,),
        in_specs=[
            pl.BlockSpec((b // xpose_grid, d), lambda g: (g, 0)),
            pl.BlockSpec((b // xpose_grid, d), lambda g: (g, 0)),
        ],
        out_specs=[
            pl.BlockSpec((d, b // xpose_grid), lambda g: (0, g)),
            pl.BlockSpec((d, b // xpose_grid), lambda g: (0, g)),
        ],
        out_shape=[
            jax.ShapeDtypeStruct((d, b), jnp.int32),
            jax.ShapeDtypeStruct((d, b), jnp.float32),
        ],
    )(index, src)

    win2 = _make_sc_winner(m, b, d)(idxt, srct)

    out = pl.pallas_call(
        _merge_body,
        grid=(merge_grid,),
        in_specs=[
            pl.BlockSpec((d * mb,), lambda g: (g,)),
            pl.BlockSpec((mb, d), lambda g: (g, 0)),
        ],
        out_specs=pl.BlockSpec((mb, d), lambda g: (g, 0)),
        out_shape=jax.ShapeDtypeStruct((m, d), jnp.float32),
        scratch_shapes=[pltpu.VMEM((d, mb), jnp.int32)],
    )(win2, self_tensor)
    return out


# speedup vs baseline: 14.5799x; 1.0040x over previous
import functools

import jax
import jax.numpy as jnp
from jax import lax
from jax.experimental import pallas as pl
from jax.experimental.pallas import tpu as pltpu
from jax.experimental.pallas import tpu_sc as plsc

_LANES = 16
_NC = 2
_NS = 16
_NW = _NC * _NS
_CH = 4096
_G1 = 32
_G2 = 16


def _xpose_body(idx_ref, src_ref, idxt_ref, srct_ref):
    idxt_ref[...] = idx_ref[...].T
    srct_ref[...] = src_ref[...].T


def _merge_body(win2_ref, self_ref, out_ref, scr_ref):
    d, mb = self_ref.shape[1], self_ref.shape[0]
    for j in range(d):
        scr_ref[j, :] = win2_ref[pl.ds(j * mb, mb)]
    wt = scr_ref[...].T
    out_ref[...] = jnp.where(wt == -1, self_ref[...],
                             jax.lax.bitcast_convert_type(wt, jnp.float32))


_NP = 10


@functools.cache
def _make_sc_winner(m, b, d):
    cpw = d // _NW
    n_chunks = b // _CH
    g1_iters = _CH // (_LANES * _G1)
    g2_iters = _CH // (_LANES * _G2)
    m_vregs = m // _LANES
    pm = m // _NP
    mesh = plsc.VectorSubcoreMesh(core_axis_name="c", subcore_axis_name="s")

    @functools.partial(
        pl.kernel,
        out_type=jax.ShapeDtypeStruct((_NP * d * pm,), jnp.int32),
        mesh=mesh,
        compiler_params=pltpu.CompilerParams(needs_layout_passes=False),
        scratch_types=[
            pltpu.VMEM((m,), jnp.int32),
            pltpu.VMEM((_CH,), jnp.int32),
            pltpu.VMEM((_CH,), jnp.int32),
            pltpu.VMEM((_CH,), jnp.float32),
            pltpu.VMEM((_CH,), jnp.float32),
        ] + [pltpu.SemaphoreType.DMA for _ in range(4 + _NP)],
    )
    def sc_winner(idxt_hbm, srct_hbm, win2_hbm,
                  win_ref, idx_a, idx_b, src_a, src_b, *sems):
        sem_i0, sem_i1, sem_v0, sem_v1 = sems[:4]
        sem_w = sems[4:]
        idx_bufs = (idx_a, idx_b)
        src_bufs = (src_a, src_b)
        sem_i = (sem_i0, sem_i1)
        sem_v = (sem_v0, sem_v1)
        cc = lax.axis_index("c")
        ss = lax.axis_index("s")
        wid = ss * _NC + cc
        lanes = lax.iota(jnp.int32, _LANES)
        neg1 = jnp.full((_LANES,), -1, jnp.int32)

        unroll = 25

        def init_pieces(p0, p1):
            def init_group(v, _):
                for u in range(unroll):
                    win_ref[pl.ds((v * unroll + u) * _LANES, _LANES)] = neg1
                return 0

            lax.fori_loop(p0 * (pm // _LANES // unroll),
                          p1 * (pm // _LANES // unroll), init_group, 0)

        with jax.named_scope("meminit"):
            init_pieces(0, _NP)

        def column(col, _):
            j = wid * cpw + col

            def load_idx(c):
                return pltpu.async_copy(
                    idxt_hbm.at[j, pl.ds(c * _CH, _CH)],
                    idx_bufs[c % 2], sem_i[c % 2])

            def load_src(c):
                return pltpu.async_copy(
                    srct_hbm.at[j, pl.ds(c * _CH, _CH)],
                    src_bufs[c % 2], sem_v[c % 2])

            idesc = load_idx(0)

            ns1 = jax.named_scope("pass1")
            ns1.__enter__()
            for c in range(n_chunks):
                nxt = load_idx(c + 1) if c + 1 < n_chunks else None
                idesc.wait()
                cb = idx_bufs[c % 2]

                def p1_group(gi, _, c=c, cb=cb):
                    base = c * _CH + gi * (_LANES * _G1)
                    ts = []
                    ivs = []
                    for v in range(_G1):
                        ts.append(cb[pl.ds(gi * (_LANES * _G1) + v * _LANES,
                                           _LANES)])
                        ivs.append(base + v * _LANES + lanes)
                    for v in range(_G1):
                        plsc.store_scatter(win_ref, [ts[v]], ivs[v])
                    bad = None
                    for v in range(_G1):
                        w = plsc.load_gather(win_ref, [ts[v]])
                        mv = w < ivs[v]
                        bad = mv if bad is None else (bad | mv)

                    def fix_cond(nbad):
                        return nbad > 0

                    def fix_body(nbad):
                        accv = jnp.zeros((_LANES,), jnp.int32)
                        for v in range(_G1):
                            w = plsc.load_gather(win_ref, [ts[v]])
                            plsc.store_scatter(win_ref, [ts[v]], ivs[v],
                                               mask=w < ivs[v])
                        for v in range(_G1):
                            w = plsc.load_gather(win_ref, [ts[v]])
                            accv = accv + (w < ivs[v]).astype(jnp.int32)
                        return jnp.sum(accv)

                    lax.while_loop(fix_cond, fix_body,
                                   jnp.any(bad).astype(jnp.int32))
                    return 0

                lax.fori_loop(0, g1_iters, p1_group, 0)
                idesc = nxt
            ns1.__exit__(None, None, None)

            ns2 = jax.named_scope("pass2")
            ns2.__enter__()
            idesc = load_idx(0)
            vdesc = load_src(0)
            for c in range(n_chunks):
                nxti = load_idx(c + 1) if c + 1 < n_chunks else None
                nxtv = load_src(c + 1) if c + 1 < n_chunks else None
                idesc.wait()
                vdesc.wait()
                cb = idx_bufs[c % 2]
                vb = src_bufs[c % 2]

                def p2_group(gi, _, c=c, cb=cb, vb=vb):
                    base = c * _CH + gi * (_LANES * _G2)
                    ts, masks, vals = [], [], []
                    for v in range(_G2):
                        sl = pl.ds(gi * (_LANES * _G2) + v * _LANES, _LANES)
                        t = cb[sl]
                        iv = base + v * _LANES + lanes
                        w = plsc.load_gather(win_ref, [t])
                        ts.append(t)
                        masks.append(w == iv)
                        vals.append(plsc.bitcast(vb[sl], jnp.int32))
                    for v in range(_G2):
                        plsc.store_scatter(win_ref, [ts[v]], vals[v],
                                           mask=masks[v])
                    return 0

                lax.fori_loop(0, g2_iters, p2_group, 0)
                idesc = nxti
                vdesc = nxtv
            ns2.__exit__(None, None, None)

            with jax.named_scope("win_out"):
                wdescs = [
                    pltpu.async_copy(win_ref.at[pl.ds(p * pm, pm)],
                                     win2_hbm.at[pl.ds((p * d + j) * pm, pm)],
                                     sem_w[p])
                    for p in range(_NP)
                ]
                for p, wd in enumerate(wdescs):
                    wd.wait()
                    init_pieces(p, p + 1)
            return 0

        lax.fori_loop(0, cpw, column, 0)

    return sc_winner


def kernel(self_tensor, index, src):
    m, d = self_tensor.shape
    b = index.shape[0]
    xpose_grid = 8
    merge_grid = _NP
    mb = m // merge_grid

    idxt, srct = pl.pallas_call(
        _xpose_body,
        grid=(xpose_grid,),
        in_specs=[
            pl.BlockSpec((b // xpose_grid, d), lambda g: (g, 0)),
            pl.BlockSpec((b // xpose_grid, d), lambda g: (g, 0)),
        ],
        out_specs=[
            pl.BlockSpec((d, b // xpose_grid), lambda g: (0, g)),
            pl.BlockSpec((d, b // xpose_grid), lambda g: (0, g)),
        ],
        out_shape=[
            jax.ShapeDtypeStruct((d, b), jnp.int32),
            jax.ShapeDtypeStruct((d, b), jnp.float32),
        ],
    )(index, src)

    win2 = _make_sc_winner(m, b, d)(idxt, srct)

    out = pl.pallas_call(
        _merge_body,
        grid=(merge_grid,),
        in_specs=[
            pl.BlockSpec((d * mb,), lambda g: (g,)),
            pl.BlockSpec((mb, d), lambda g: (g, 0)),
        ],
        out_specs=pl.BlockSpec((mb, d), lambda g: (g, 0)),
        out_shape=jax.ShapeDtypeStruct((m, d), jnp.float32),
        scratch_shapes=[pltpu.VMEM((d, mb), jnp.int32)],
    )(win2, self_tensor)
    return out
